# baseline (speedup 1.0000x reference)
import jax
import jax.numpy as jnp
from jax.experimental import pallas as pl
from jax.experimental.pallas import tpu as pltpu

_SPLIT = 6


def _copy_kernel(x_ref, o_ref, buf0, buf1, si0, si1, so0, so1):
    t = x_ref.shape[0]
    in0 = pltpu.make_async_copy(x_ref.at[pl.ds(0, _SPLIT)], buf0, si0)
    in1 = pltpu.make_async_copy(x_ref.at[pl.ds(_SPLIT, t - _SPLIT)], buf1, si1)
    in0.start()
    in1.start()
    in0.wait()
    out0 = pltpu.make_async_copy(buf0, o_ref.at[pl.ds(0, _SPLIT)], so0)
    out0.start()
    in1.wait()
    out1 = pltpu.make_async_copy(buf1, o_ref.at[pl.ds(_SPLIT, t - _SPLIT)], so1)
    out1.start()
    out0.wait()
    out1.wait()


def kernel(input):
    t, b, s = input.shape
    return pl.pallas_call(
        _copy_kernel,
        out_shape=jax.ShapeDtypeStruct(input.shape, input.dtype),
        in_specs=[pl.BlockSpec(memory_space=pl.ANY)],
        out_specs=pl.BlockSpec(memory_space=pl.ANY),
        scratch_shapes=[
            pltpu.VMEM((_SPLIT, b, s), input.dtype),
            pltpu.VMEM((t - _SPLIT, b, s), input.dtype),
            pltpu.SemaphoreType.DMA,
            pltpu.SemaphoreType.DMA,
            pltpu.SemaphoreType.DMA,
            pltpu.SemaphoreType.DMA,
        ],
    )(input)
